# Initial kernel scaffold; baseline (speedup 1.0000x reference)
import jax
import jax.numpy as jnp
from jax.experimental import pallas as pl


def kernel(x, edge_index, W1, b1, W2, b2, Wfc, bfc):
    raise NotImplementedError("write your pallas kernel here")


# speedup vs baseline: 19.7516x; 19.7516x over previous
import functools

import jax
import jax.numpy as jnp
from jax import lax
from jax.experimental import pallas as pl
from jax.experimental.pallas import tpu as pltpu
from jax.experimental.pallas import tpu_sc as plsc

N = 10000
E = 320000
D_IN = 128
D_H = 256
D_OUT = 128

NC = 2
NS = 16
K = 128
CHUNKS = 2560
E_PAD = CHUNKS * K
CPC = CHUNKS // NC
G = 32
NP = 10240
RPT = NP // NS

_MESH = plsc.VectorSubcoreMesh(
    core_axis_name="c", subcore_axis_name="s", num_cores=NC, num_subcores=NS
)


def _f32(*shape):
    return jax.ShapeDtypeStruct(shape, jnp.float32)


@functools.partial(
    pl.kernel,
    out_type=(_f32(NP, 128), _f32(NP, 128)),
    mesh=_MESH,
    scratch_types=[
        pltpu.VMEM_SHARED((NP, 128), jnp.float32),
        pltpu.VMEM((K, 128), jnp.float32),
        pltpu.VMEM((G, K), jnp.int32),
    ],
)
def _deg_kernel(dst_hbm, ones_hbm, zeros_hbm, p0_hbm, p1_hbm, acc, ones_v, dst_v):
    c = lax.axis_index("c")
    s = lax.axis_index("s")
    row0 = s * RPT
    tile_chunks = CPC // NS
    base = c * CPC + s * tile_chunks
    pltpu.sync_copy(zeros_hbm.at[pl.ds(row0, RPT)], acc.at[pl.ds(row0, RPT)])
    pltpu.sync_copy(ones_hbm, ones_v)
    plsc.subcore_barrier()

    def group(gi, _):
        pltpu.sync_copy(dst_hbm.at[pl.ds(base + gi * G, G)], dst_v)

        def body(j, _):
            pltpu.sync_copy(ones_v, acc.at[dst_v.at[j]], add=True)
            return ()

        lax.fori_loop(0, G, body, (), unroll=False)
        return ()

    lax.fori_loop(0, tile_chunks // G, group, (), unroll=False)
    plsc.subcore_barrier()

    @pl.when(c == 0)
    def _():
        pltpu.sync_copy(acc.at[pl.ds(row0, RPT)], p0_hbm.at[pl.ds(row0, RPT)])

    @pl.when(c == 1)
    def _():
        pltpu.sync_copy(acc.at[pl.ds(row0, RPT)], p1_hbm.at[pl.ds(row0, RPT)])


@functools.partial(
    pl.kernel,
    out_type=(_f32(NP, 128), _f32(NP, 128)),
    mesh=_MESH,
    scratch_types=[
        pltpu.VMEM_SHARED((NP, 128), jnp.float32),
        pltpu.VMEM((G, K), jnp.int32),
        pltpu.VMEM((G, K), jnp.int32),
        pltpu.VMEM((K, 128), jnp.float32),
        pltpu.VMEM((K, 128), jnp.float32),
        pltpu.SemaphoreType.DMA,
        pltpu.SemaphoreType.DMA,
    ],
)
def _agg1_kernel(table_hbm, src_hbm, dst_hbm, zeros_hbm, q0_hbm, q1_hbm,
                 acc, src_v, dst_v, gbuf_a, gbuf_b, sem_a, sem_b):
    c = lax.axis_index("c")
    s = lax.axis_index("s")
    row0 = s * RPT
    tile_chunks = CPC // NS
    base = c * CPC + s * tile_chunks
    pltpu.sync_copy(zeros_hbm.at[pl.ds(row0, RPT)], acc.at[pl.ds(row0, RPT)])
    plsc.subcore_barrier()

    def group(gi, _):
        pltpu.sync_copy(src_hbm.at[pl.ds(base + gi * G, G)], src_v)
        pltpu.sync_copy(dst_hbm.at[pl.ds(base + gi * G, G)], dst_v)

        def pair(j, _):
            da = pltpu.async_copy(table_hbm.at[src_v.at[2 * j]], gbuf_a, sem_a)
            db = pltpu.async_copy(table_hbm.at[src_v.at[2 * j + 1]], gbuf_b, sem_b)
            da.wait()
            pltpu.sync_copy(gbuf_a, acc.at[dst_v.at[2 * j]], add=True)
            db.wait()
            pltpu.sync_copy(gbuf_b, acc.at[dst_v.at[2 * j + 1]], add=True)
            return ()

        lax.fori_loop(0, G // 2, pair, (), unroll=False)
        return ()

    lax.fori_loop(0, tile_chunks // G, group, (), unroll=False)
    plsc.subcore_barrier()

    @pl.when(c == 0)
    def _():
        pltpu.sync_copy(acc.at[pl.ds(row0, RPT)], q0_hbm.at[pl.ds(row0, RPT)])

    @pl.when(c == 1)
    def _():
        pltpu.sync_copy(acc.at[pl.ds(row0, RPT)], q1_hbm.at[pl.ds(row0, RPT)])


@functools.partial(
    pl.kernel,
    out_type=(_f32(NP, 128), _f32(NP, 128)),
    mesh=_MESH,
    scratch_types=[
        pltpu.VMEM_SHARED((NP, 128), jnp.float32),
        pltpu.VMEM((G, K), jnp.int32),
        pltpu.VMEM((G, K), jnp.int32),
        pltpu.VMEM((K, 128), jnp.float32),
        pltpu.VMEM((K, 128), jnp.float32),
        pltpu.SemaphoreType.DMA,
        pltpu.SemaphoreType.DMA,
    ],
)
def _agg2_kernel(ta_hbm, tb_hbm, src_hbm, dst_hbm, zeros_hbm, r0_hbm, r1_hbm,
                 acc, src_v, dst_v, gbuf_a, gbuf_b, sem_a, sem_b):
    c = lax.axis_index("c")
    s = lax.axis_index("s")
    row0 = s * RPT
    tile_chunks = CHUNKS // NS
    base = s * tile_chunks
    pltpu.sync_copy(zeros_hbm.at[pl.ds(row0, RPT)], acc.at[pl.ds(row0, RPT)])
    plsc.subcore_barrier()

    def make_group(table):
        def group(gi, _):
            pltpu.sync_copy(src_hbm.at[pl.ds(base + gi * G, G)], src_v)
            pltpu.sync_copy(dst_hbm.at[pl.ds(base + gi * G, G)], dst_v)

            def pair(j, _):
                da = pltpu.async_copy(table.at[src_v.at[2 * j]], gbuf_a, sem_a)
                db = pltpu.async_copy(table.at[src_v.at[2 * j + 1]], gbuf_b, sem_b)
                da.wait()
                pltpu.sync_copy(gbuf_a, acc.at[dst_v.at[2 * j]], add=True)
                db.wait()
                pltpu.sync_copy(gbuf_b, acc.at[dst_v.at[2 * j + 1]], add=True)
                return ()

            lax.fori_loop(0, G // 2, pair, (), unroll=False)
            return ()

        return group

    @pl.when(c == 0)
    def _():
        lax.fori_loop(0, tile_chunks // G, make_group(ta_hbm), (), unroll=False)

    @pl.when(c == 1)
    def _():
        lax.fori_loop(0, tile_chunks // G, make_group(tb_hbm), (), unroll=False)

    plsc.subcore_barrier()

    @pl.when(c == 0)
    def _():
        pltpu.sync_copy(acc.at[pl.ds(row0, RPT)], r0_hbm.at[pl.ds(row0, RPT)])

    @pl.when(c == 1)
    def _():
        pltpu.sync_copy(acc.at[pl.ds(row0, RPT)], r1_hbm.at[pl.ds(row0, RPT)])


BLK = 400
GRID = N // BLK


def _dinv_of(p0, p1):
    deg = 1.0 + p0[:, 0:1] + p1[:, 0:1]
    return lax.rsqrt(deg)


def _k2_body(x_ref, p0_ref, p1_ref, xp_ref):
    dinv = _dinv_of(p0_ref[...], p1_ref[...])
    xp_ref[...] = x_ref[...] * dinv


def _k4_body(q0_ref, q1_ref, xp_ref, p0_ref, p1_ref, w1_ref, b1_ref,
             ha_ref, hb_ref):
    dinv = _dinv_of(p0_ref[...], p1_ref[...])
    u = dinv * (q0_ref[...] + q1_ref[...] + xp_ref[...])
    h1 = jnp.maximum(
        jnp.dot(u, w1_ref[...], preferred_element_type=jnp.float32)
        + b1_ref[...], 0.0)
    h1s = dinv * h1
    ha_ref[...] = h1s[:, :128]
    hb_ref[...] = h1s[:, 128:]


def _k6_body(r0_ref, r1_ref, ha_ref, hb_ref, p0_ref, p1_ref,
             w2_ref, b2_ref, wfc_ref, bfc_ref, out_ref):
    dinv = _dinv_of(p0_ref[...], p1_ref[...])
    va = dinv * (r0_ref[...] + ha_ref[...])
    vb = dinv * (r1_ref[...] + hb_ref[...])
    v = jnp.concatenate([va, vb], axis=1)
    h2 = jnp.maximum(
        jnp.dot(v, w2_ref[...], preferred_element_type=jnp.float32)
        + b2_ref[...], 0.0)
    out_ref[...] = (
        jnp.dot(h2, wfc_ref[...], preferred_element_type=jnp.float32)
        + bfc_ref[...])


def _row_spec(cols):
    return pl.BlockSpec((BLK, cols), lambda i: (i, 0))


def _full_spec(r, cols):
    return pl.BlockSpec((r, cols), lambda i: (0, 0))


def kernel(x, edge_index, W1, b1, W2, b2, Wfc, bfc):
    npad = E_PAD - E
    pad_src = (jnp.arange(npad, dtype=jnp.int32) * 37) % N
    pad_dst = N + (jnp.arange(npad, dtype=jnp.int32) % (NP - N))
    src2d = jnp.concatenate([edge_index[0], pad_src]).reshape(CHUNKS, K)
    dst2d = jnp.concatenate([edge_index[1], pad_dst]).reshape(CHUNKS, K)
    ones128 = jnp.ones((K, 128), jnp.float32)
    zeros128 = jnp.zeros((NP, 128), jnp.float32)

    p0, p1 = _deg_kernel(dst2d, ones128, zeros128)

    xp = pl.pallas_call(
        _k2_body,
        grid=(GRID,),
        in_specs=[_row_spec(128), _row_spec(128), _row_spec(128)],
        out_specs=_row_spec(128),
        out_shape=_f32(N, 128),
    )(x, p0, p1)

    q0, q1 = _agg1_kernel(xp, src2d, dst2d, zeros128)

    h1a, h1b = pl.pallas_call(
        _k4_body,
        grid=(GRID,),
        in_specs=[_row_spec(128), _row_spec(128), _row_spec(128),
                  _row_spec(128), _row_spec(128),
                  _full_spec(128, 256), _full_spec(1, 256)],
        out_specs=[_row_spec(128), _row_spec(128)],
        out_shape=[_f32(N, 128), _f32(N, 128)],
    )(q0, q1, xp, p0, p1, W1, b1.reshape(1, D_H))

    r0, r1 = _agg2_kernel(h1a, h1b, src2d, dst2d, zeros128)

    out = pl.pallas_call(
        _k6_body,
        grid=(GRID,),
        in_specs=[_row_spec(128), _row_spec(128), _row_spec(128),
                  _row_spec(128), _row_spec(128), _row_spec(128),
                  _full_spec(256, 256), _full_spec(1, 256),
                  _full_spec(256, 128), _full_spec(1, 128)],
        out_specs=_row_spec(128),
        out_shape=_f32(N, D_OUT),
    )(r0, r1, h1a, h1b, p0, p1, W2, b2.reshape(1, D_H),
      Wfc, bfc.reshape(1, D_OUT))
    return out
